# baseline (speedup 1.0000x reference)
import functools

import numpy as np
import jax
import jax.numpy as jnp
from jax import lax
from jax.experimental import pallas as pl
from jax.experimental.pallas import tpu as pltpu
from jax.experimental.pallas import tpu_sc as plsc

N = 10000
E = 320000
D = 128
NUM_LAYERS = 2
ALPHA = 0.5
THETA = 1.0

NC = 2
NS = 16
NW = NC * NS
CH = 112
K0 = 168
K1 = 12
NBUF = 3
TCH = NS * (K0 + K1)
EP = TCH * CH
SINK = N
ACC_ROWS = 10112
RPT = ACC_ROWS // NS


def _make_agg():
    mesh = plsc.VectorSubcoreMesh(core_axis_name="c", subcore_axis_name="s")

    @functools.partial(
        pl.kernel,
        mesh=mesh,
        out_type=jax.ShapeDtypeStruct((NC, ACC_ROWS, D), jnp.float32),
        scratch_types=[
            pltpu.VMEM((2 * NBUF, 2, CH), jnp.int32),
            pltpu.VMEM((NBUF, CH, D), jnp.float32),
            pltpu.VMEM_SHARED((ACC_ROWS, D), jnp.float32),
            pltpu.SemaphoreType.DMA((2 * NBUF,)),
            pltpu.SemaphoreType.DMA((NBUF,)),
            pltpu.SemaphoreType.DMA((NBUF,)),
        ],
    )
    def agg(h_hbm, idx_hbm, out_hbm,
            idx_v, rows_v, acc, isem, gsem, ssem):
        cid = lax.axis_index("c")
        sid = lax.axis_index("s")
        row0 = sid * RPT
        KC = jnp.where(cid == 0, K0, K1)
        c0 = cid * (NS * K0) + sid * KC

        NI = 2 * NBUF

        def start_idx(k, i):
            pltpu.async_copy(idx_hbm.at[c0 + k], idx_v.at[i], isem.at[i])

        def wait_idx(i):
            pltpu.make_async_copy(idx_hbm.at[0], idx_v.at[i],
                                  isem.at[i]).wait()

        def start_gather(i, b):
            pltpu.async_copy(h_hbm.at[idx_v.at[i, 0]], rows_v.at[b],
                             gsem.at[b])

        def wait_gather(b):
            pltpu.make_async_copy(h_hbm.at[idx_v.at[0, 0]], rows_v.at[b],
                                  gsem.at[b]).wait()

        def start_scatter(i, b):
            pltpu.async_copy(rows_v.at[b], acc.at[idx_v.at[i, 1]],
                             ssem.at[b], add=True)

        def wait_scatter(b):
            pltpu.make_async_copy(rows_v.at[b], acc.at[idx_v.at[0, 1]],
                                  ssem.at[b]).wait()

        for m in range(NI - 1):
            start_idx(m, m)

        zvec = jnp.zeros((16,), jnp.float32)

        def zrow(r, carry):
            for c in range(D // 16):
                rows_v[0, r, pl.ds(c * 16, 16)] = zvec
            return carry

        lax.fori_loop(0, CH, zrow, 0)
        for q in range(RPT // CH):
            pltpu.sync_copy(rows_v.at[0],
                            acc.at[pl.ds(row0 + q * CH, CH)])
        TAIL = RPT % CH
        if TAIL:
            pltpu.sync_copy(rows_v.at[0, pl.ds(0, TAIL)],
                            acc.at[pl.ds(row0 + RPT - TAIL, TAIL)])

        for m in range(NBUF - 1):
            wait_idx(m)
            start_gather(m, m)
        plsc.subcore_barrier()

        def body(j, carry):
            k0 = j * NBUF
            for b in range(NBUF):
                k = k0 + b
                bp = (b - 1) % NBUF
                wait_gather(b)
                start_scatter(k % NI, b)

                @pl.when(k >= 1)
                def _():
                    wait_scatter(bp)

                @pl.when(k + NI - 1 < KC)
                def _():
                    start_idx(k + NI - 1, (k + NI - 1) % NI)

                @pl.when(k + NBUF - 1 < KC)
                def _():
                    wait_idx((k + NBUF - 1) % NI)
                    start_gather((k + NBUF - 1) % NI, bp)

            return carry

        lax.fori_loop(0, KC // NBUF, body, 0)
        wait_scatter((K0 - 1) % NBUF)

        plsc.subcore_barrier()
        pltpu.sync_copy(acc.at[pl.ds(row0, RPT)],
                        out_hbm.at[cid, pl.ds(row0, RPT)])

    return agg


_AGG = _make_agg()


BR = 1000


def _mm_body(x_ref, w_ref, o_ref):
    o_ref[...] = jnp.dot(x_ref[...], w_ref[...],
                         preferred_element_type=jnp.float32)


def _matmul(x, w):
    return pl.pallas_call(
        _mm_body,
        grid=(N // BR,),
        in_specs=[pl.BlockSpec((BR, D), lambda i: (i, 0)),
                  pl.BlockSpec((D, D), lambda i: (0, 0))],
        out_specs=pl.BlockSpec((BR, D), lambda i: (i, 0)),
        out_shape=jax.ShapeDtypeStruct((N, D), jnp.float32),
    )(x, w)


def _bias_body(do_relu, p_ref, b_ref, o_ref):
    s = p_ref[0] + p_ref[1] + b_ref[...]
    o_ref[...] = jnp.maximum(s, 0.0) if do_relu else s


def _bias_combine(p, b, do_relu):
    return pl.pallas_call(
        functools.partial(_bias_body, do_relu),
        grid=(N // BR,),
        in_specs=[pl.BlockSpec((NC, BR, D), lambda i: (0, i, 0)),
                  pl.BlockSpec((1, D), lambda i: (0, 0))],
        out_specs=pl.BlockSpec((BR, D), lambda i: (i, 0)),
        out_shape=jax.ShapeDtypeStruct((N, D), jnp.float32),
    )(p, b.reshape(1, D))


def _gcn2_body(beta, has_w2, p_ref, x0_ref, w_ref, w2_ref, o_ref):
    comb = (p_ref[0] + p_ref[1]) * (1.0 - ALPHA) + ALPHA * x0_ref[...]
    h = (1.0 - beta) * comb + beta * jnp.dot(
        comb, w_ref[...], preferred_element_type=jnp.float32)
    h = jnp.maximum(h, 0.0)
    if has_w2:
        h = jnp.dot(h, w2_ref[...], preferred_element_type=jnp.float32)
    o_ref[...] = h


def _gcn2_combine(p, x0, w, beta, w2=None):
    has_w2 = w2 is not None
    return pl.pallas_call(
        functools.partial(_gcn2_body, beta, has_w2),
        grid=(N // BR,),
        in_specs=[pl.BlockSpec((NC, BR, D), lambda i: (0, i, 0)),
                  pl.BlockSpec((BR, D), lambda i: (i, 0)),
                  pl.BlockSpec((D, D), lambda i: (0, 0)),
                  pl.BlockSpec((D, D), lambda i: (0, 0))],
        out_specs=pl.BlockSpec((BR, D), lambda i: (i, 0)),
        out_shape=jax.ShapeDtypeStruct((N, D), jnp.float32),
    )(p, x0, w, w2 if has_w2 else w)


def kernel(x, edge_index, W_in, b_in, W_layers, W_out, b_out):
    src = edge_index[0]
    dst = edge_index[1]
    pad = EP - E
    src_p = jnp.concatenate([src, jnp.zeros((pad,), jnp.int32)]
                            ).reshape(TCH, 1, CH)
    dst_p = jnp.concatenate([dst, jnp.full((pad,), SINK, jnp.int32)]
                            ).reshape(TCH, 1, CH)
    idx_p = jnp.concatenate([src_p, dst_p], axis=1)

    h = _matmul(x, W_in)
    p = _AGG(h, idx_p)
    x0 = _bias_combine(p, b_in, do_relu=True)

    p = _AGG(x0, idx_p)
    beta0 = float(np.log(THETA + 1.0))
    h = _gcn2_combine(p, x0, W_layers[0], beta0)

    p = _AGG(h, idx_p)
    beta1 = float(np.log(THETA / 2.0 + 1.0))
    h = _gcn2_combine(p, x0, W_layers[1], beta1, w2=W_out)

    p = _AGG(h, idx_p)
    return _bias_combine(p, b_out, do_relu=False)


# speedup vs baseline: 1.0104x; 1.0104x over previous
import functools

import numpy as np
import jax
import jax.numpy as jnp
from jax import lax
from jax.experimental import pallas as pl
from jax.experimental.pallas import tpu as pltpu
from jax.experimental.pallas import tpu_sc as plsc

N = 10000
E = 320000
D = 128
NUM_LAYERS = 2
ALPHA = 0.5
THETA = 1.0

NC = 2
NS = 16
NW = NC * NS
CH = 112
K0 = 162
K1 = 18
NBUF = 3
TCH = NS * (K0 + K1)
EP = TCH * CH
SINK = N
ACC_ROWS = 10112
RPT = ACC_ROWS // NS


def _make_agg():
    mesh = plsc.VectorSubcoreMesh(core_axis_name="c", subcore_axis_name="s")

    @functools.partial(
        pl.kernel,
        mesh=mesh,
        out_type=jax.ShapeDtypeStruct((NC, ACC_ROWS, D), jnp.float32),
        scratch_types=[
            pltpu.VMEM((2 * NBUF, 2, CH), jnp.int32),
            pltpu.VMEM((NBUF, CH, D), jnp.float32),
            pltpu.VMEM_SHARED((ACC_ROWS, D), jnp.float32),
            pltpu.SemaphoreType.DMA((2 * NBUF,)),
            pltpu.SemaphoreType.DMA((NBUF,)),
            pltpu.SemaphoreType.DMA((NBUF,)),
        ],
    )
    def agg(h_hbm, idx_hbm, out_hbm,
            idx_v, rows_v, acc, isem, gsem, ssem):
        cid = lax.axis_index("c")
        sid = lax.axis_index("s")
        row0 = sid * RPT
        KC = jnp.where(cid == 0, K0, K1)
        c0 = cid * (NS * K0) + sid * KC

        NI = 2 * NBUF

        def start_idx(k, i):
            pltpu.async_copy(idx_hbm.at[c0 + k], idx_v.at[i], isem.at[i])

        def wait_idx(i):
            pltpu.make_async_copy(idx_hbm.at[0], idx_v.at[i],
                                  isem.at[i]).wait()

        def start_gather(i, b):
            pltpu.async_copy(h_hbm.at[idx_v.at[i, 0]], rows_v.at[b],
                             gsem.at[b])

        def wait_gather(b):
            pltpu.make_async_copy(h_hbm.at[idx_v.at[0, 0]], rows_v.at[b],
                                  gsem.at[b]).wait()

        def start_scatter(i, b):
            pltpu.async_copy(rows_v.at[b], acc.at[idx_v.at[i, 1]],
                             ssem.at[b], add=True)

        def wait_scatter(b):
            pltpu.make_async_copy(rows_v.at[b], acc.at[idx_v.at[0, 1]],
                                  ssem.at[b]).wait()

        for m in range(NI - 1):
            start_idx(m, m)

        zvec = jnp.zeros((16,), jnp.float32)

        def zrow(r, carry):
            for c in range(D // 16):
                rows_v[0, r, pl.ds(c * 16, 16)] = zvec
            return carry

        lax.fori_loop(0, CH, zrow, 0)
        for q in range(RPT // CH):
            pltpu.sync_copy(rows_v.at[0],
                            acc.at[pl.ds(row0 + q * CH, CH)])
        TAIL = RPT % CH
        if TAIL:
            pltpu.sync_copy(rows_v.at[0, pl.ds(0, TAIL)],
                            acc.at[pl.ds(row0 + RPT - TAIL, TAIL)])

        for m in range(NBUF - 1):
            wait_idx(m)
            start_gather(m, m)
        plsc.subcore_barrier()

        def body(j, carry):
            k0 = j * NBUF
            for b in range(NBUF):
                k = k0 + b
                bp = (b - 1) % NBUF
                wait_gather(b)
                start_scatter(k % NI, b)

                @pl.when(k >= 1)
                def _():
                    wait_scatter(bp)

                @pl.when(k + NI - 1 < KC)
                def _():
                    start_idx(k + NI - 1, (k + NI - 1) % NI)

                @pl.when(k + NBUF - 1 < KC)
                def _():
                    wait_idx((k + NBUF - 1) % NI)
                    start_gather((k + NBUF - 1) % NI, bp)

            return carry

        lax.fori_loop(0, KC // NBUF, body, 0)
        wait_scatter((K0 - 1) % NBUF)

        plsc.subcore_barrier()
        pltpu.sync_copy(acc.at[pl.ds(row0, RPT)],
                        out_hbm.at[cid, pl.ds(row0, RPT)])

    return agg


_AGG = _make_agg()


BR = 1000


def _mm_body(x_ref, w_ref, o_ref):
    o_ref[...] = jnp.dot(x_ref[...], w_ref[...],
                         preferred_element_type=jnp.float32)


def _matmul(x, w):
    return pl.pallas_call(
        _mm_body,
        grid=(N // BR,),
        in_specs=[pl.BlockSpec((BR, D), lambda i: (i, 0)),
                  pl.BlockSpec((D, D), lambda i: (0, 0))],
        out_specs=pl.BlockSpec((BR, D), lambda i: (i, 0)),
        out_shape=jax.ShapeDtypeStruct((N, D), jnp.float32),
    )(x, w)


def _bias_body(do_relu, p_ref, b_ref, o_ref):
    s = p_ref[0] + p_ref[1] + b_ref[...]
    o_ref[...] = jnp.maximum(s, 0.0) if do_relu else s


def _bias_combine(p, b, do_relu):
    return pl.pallas_call(
        functools.partial(_bias_body, do_relu),
        grid=(N // BR,),
        in_specs=[pl.BlockSpec((NC, BR, D), lambda i: (0, i, 0)),
                  pl.BlockSpec((1, D), lambda i: (0, 0))],
        out_specs=pl.BlockSpec((BR, D), lambda i: (i, 0)),
        out_shape=jax.ShapeDtypeStruct((N, D), jnp.float32),
    )(p, b.reshape(1, D))


def _gcn2_body(beta, has_w2, p_ref, x0_ref, w_ref, w2_ref, o_ref):
    comb = (p_ref[0] + p_ref[1]) * (1.0 - ALPHA) + ALPHA * x0_ref[...]
    h = (1.0 - beta) * comb + beta * jnp.dot(
        comb, w_ref[...], preferred_element_type=jnp.float32)
    h = jnp.maximum(h, 0.0)
    if has_w2:
        h = jnp.dot(h, w2_ref[...], preferred_element_type=jnp.float32)
    o_ref[...] = h


def _gcn2_combine(p, x0, w, beta, w2=None):
    has_w2 = w2 is not None
    return pl.pallas_call(
        functools.partial(_gcn2_body, beta, has_w2),
        grid=(N // BR,),
        in_specs=[pl.BlockSpec((NC, BR, D), lambda i: (0, i, 0)),
                  pl.BlockSpec((BR, D), lambda i: (i, 0)),
                  pl.BlockSpec((D, D), lambda i: (0, 0)),
                  pl.BlockSpec((D, D), lambda i: (0, 0))],
        out_specs=pl.BlockSpec((BR, D), lambda i: (i, 0)),
        out_shape=jax.ShapeDtypeStruct((N, D), jnp.float32),
    )(p, x0, w, w2 if has_w2 else w)


def kernel(x, edge_index, W_in, b_in, W_layers, W_out, b_out):
    src = edge_index[0]
    dst = edge_index[1]
    pad = EP - E
    src_p = jnp.concatenate([src, jnp.zeros((pad,), jnp.int32)]
                            ).reshape(TCH, 1, CH)
    dst_p = jnp.concatenate([dst, jnp.full((pad,), SINK, jnp.int32)]
                            ).reshape(TCH, 1, CH)
    idx_p = jnp.concatenate([src_p, dst_p], axis=1)

    h = _matmul(x, W_in)
    p = _AGG(h, idx_p)
    x0 = _bias_combine(p, b_in, do_relu=True)

    p = _AGG(x0, idx_p)
    beta0 = float(np.log(THETA + 1.0))
    h = _gcn2_combine(p, x0, W_layers[0], beta0)

    p = _AGG(h, idx_p)
    beta1 = float(np.log(THETA / 2.0 + 1.0))
    h = _gcn2_combine(p, x0, W_layers[1], beta1, w2=W_out)

    p = _AGG(h, idx_p)
    return _bias_combine(p, b_out, do_relu=False)


# speedup vs baseline: 1.0269x; 1.0164x over previous
import functools

import numpy as np
import jax
import jax.numpy as jnp
from jax import lax
from jax.experimental import pallas as pl
from jax.experimental.pallas import tpu as pltpu
from jax.experimental.pallas import tpu_sc as plsc

N = 10000
E = 320000
D = 128
NUM_LAYERS = 2
ALPHA = 0.5
THETA = 1.0

NC = 2
NS = 16
NW = NC * NS
CH = 112
K0 = 162
K1 = 18
NBUF = 3
TCH = NS * (K0 + K1)
EP = TCH * CH
SINK = N
ACC_ROWS = 10112
RPT = ACC_ROWS // NS


def _make_agg():
    mesh = plsc.VectorSubcoreMesh(core_axis_name="c", subcore_axis_name="s")

    @functools.partial(
        pl.kernel,
        mesh=mesh,
        out_type=jax.ShapeDtypeStruct((NC, ACC_ROWS, D), jnp.float32),
        scratch_types=[
            pltpu.VMEM((2 * NBUF, 2, CH), jnp.int32),
            pltpu.VMEM((NBUF, CH, D), jnp.float32),
            pltpu.VMEM_SHARED((ACC_ROWS, D), jnp.float32),
            pltpu.SemaphoreType.DMA((2 * NBUF,)),
            pltpu.SemaphoreType.DMA((NBUF,)),
            pltpu.SemaphoreType.DMA((NBUF,)),
        ],
    )
    def agg(h_hbm, idx_hbm, out_hbm,
            idx_v, rows_v, acc, isem, gsem, ssem):
        cid = lax.axis_index("c")
        sid = lax.axis_index("s")
        row0 = sid * RPT
        KC = jnp.where(cid == 0, K0, K1)
        c0 = cid * (NS * K0) + sid * KC

        NI = 2 * NBUF

        def start_idx(k, i):
            pltpu.async_copy(idx_hbm.at[c0 + k], idx_v.at[i], isem.at[i])

        def wait_idx(i):
            pltpu.make_async_copy(idx_hbm.at[0], idx_v.at[i],
                                  isem.at[i]).wait()

        def start_gather(i, b):
            pltpu.async_copy(h_hbm.at[idx_v.at[i, 0]], rows_v.at[b],
                             gsem.at[b])

        def wait_gather(b):
            pltpu.make_async_copy(h_hbm.at[idx_v.at[0, 0]], rows_v.at[b],
                                  gsem.at[b]).wait()

        def start_scatter(i, b):
            pltpu.async_copy(rows_v.at[b], acc.at[idx_v.at[i, 1]],
                             ssem.at[b], add=True)

        def wait_scatter(b):
            pltpu.make_async_copy(rows_v.at[b], acc.at[idx_v.at[0, 1]],
                                  ssem.at[b]).wait()

        for m in range(NI - 1):
            start_idx(m, m)

        zvec = jnp.zeros((16,), jnp.float32)

        def zrow(r, carry):
            for c in range(D // 16):
                rows_v[0, r, pl.ds(c * 16, 16)] = zvec
            return carry

        lax.fori_loop(0, CH, zrow, 0)
        for q in range(RPT // CH):
            pltpu.sync_copy(rows_v.at[0],
                            acc.at[pl.ds(row0 + q * CH, CH)])
        TAIL = RPT % CH
        if TAIL:
            pltpu.sync_copy(rows_v.at[0, pl.ds(0, TAIL)],
                            acc.at[pl.ds(row0 + RPT - TAIL, TAIL)])

        for m in range(NBUF - 1):
            wait_idx(m)
            start_gather(m, m)
        plsc.subcore_barrier()

        def body(j, carry):
            k0 = j * NBUF
            for b in range(NBUF):
                k = k0 + b
                bp = (b - 1) % NBUF
                wait_gather(b)
                start_scatter(k % NI, b)

                @pl.when(k >= 1)
                def _():
                    wait_scatter(bp)

                @pl.when(k + NI - 1 < KC)
                def _():
                    start_idx(k + NI - 1, (k + NI - 1) % NI)

                @pl.when(k + NBUF - 1 < KC)
                def _():
                    wait_idx((k + NBUF - 1) % NI)
                    start_gather((k + NBUF - 1) % NI, bp)

            return carry

        lax.fori_loop(0, KC // NBUF, body, 0)
        wait_scatter((K0 - 1) % NBUF)

        plsc.subcore_barrier()
        pltpu.sync_copy(acc.at[pl.ds(row0, RPT)],
                        out_hbm.at[cid, pl.ds(row0, RPT)])

    return agg


_AGG = _make_agg()


BR = 1000


def _bias_body(do_relu, has_w, p_ref, b_ref, w_ref, o_ref):
    s = p_ref[0] + p_ref[1]
    if has_w:
        s = jnp.dot(s, w_ref[...], preferred_element_type=jnp.float32)
    s = s + b_ref[...]
    o_ref[...] = jnp.maximum(s, 0.0) if do_relu else s


def _bias_combine(p, b, do_relu, w=None):
    has_w = w is not None
    return pl.pallas_call(
        functools.partial(_bias_body, do_relu, has_w),
        grid=(N // BR,),
        in_specs=[pl.BlockSpec((NC, BR, D), lambda i: (0, i, 0)),
                  pl.BlockSpec((1, D), lambda i: (0, 0)),
                  pl.BlockSpec((D, D), lambda i: (0, 0))],
        out_specs=pl.BlockSpec((BR, D), lambda i: (i, 0)),
        out_shape=jax.ShapeDtypeStruct((N, D), jnp.float32),
    )(p, b.reshape(1, D), w if has_w else jnp.zeros((D, D), jnp.float32))


def _gcn2_body(beta, has_w2, p_ref, x0_ref, w_ref, w2_ref, o_ref):
    comb = (p_ref[0] + p_ref[1]) * (1.0 - ALPHA) + ALPHA * x0_ref[...]
    h = (1.0 - beta) * comb + beta * jnp.dot(
        comb, w_ref[...], preferred_element_type=jnp.float32)
    h = jnp.maximum(h, 0.0)
    if has_w2:
        h = jnp.dot(h, w2_ref[...], preferred_element_type=jnp.float32)
    o_ref[...] = h


def _gcn2_combine(p, x0, w, beta, w2=None):
    has_w2 = w2 is not None
    return pl.pallas_call(
        functools.partial(_gcn2_body, beta, has_w2),
        grid=(N // BR,),
        in_specs=[pl.BlockSpec((NC, BR, D), lambda i: (0, i, 0)),
                  pl.BlockSpec((BR, D), lambda i: (i, 0)),
                  pl.BlockSpec((D, D), lambda i: (0, 0)),
                  pl.BlockSpec((D, D), lambda i: (0, 0))],
        out_specs=pl.BlockSpec((BR, D), lambda i: (i, 0)),
        out_shape=jax.ShapeDtypeStruct((N, D), jnp.float32),
    )(p, x0, w, w2 if has_w2 else w)


def kernel(x, edge_index, W_in, b_in, W_layers, W_out, b_out):
    src = edge_index[0]
    dst = edge_index[1]
    pad = EP - E
    src_p = jnp.concatenate([src, jnp.zeros((pad,), jnp.int32)]
                            ).reshape(TCH, 1, CH)
    dst_p = jnp.concatenate([dst, jnp.full((pad,), SINK, jnp.int32)]
                            ).reshape(TCH, 1, CH)
    idx_p = jnp.concatenate([src_p, dst_p], axis=1)

    p = _AGG(x, idx_p)
    x0 = _bias_combine(p, b_in, do_relu=True, w=W_in)

    p = _AGG(x0, idx_p)
    beta0 = float(np.log(THETA + 1.0))
    h = _gcn2_combine(p, x0, W_layers[0], beta0)

    p = _AGG(h, idx_p)
    beta1 = float(np.log(THETA / 2.0 + 1.0))
    h = _gcn2_combine(p, x0, W_layers[1], beta1, w2=W_out)

    p = _AGG(h, idx_p)
    return _bias_combine(p, b_out, do_relu=False)
